# baseline (speedup 1.0000x reference)
import jax
import jax.numpy as jnp
from jax.experimental import pallas as pl
from jax.experimental.pallas import tpu as pltpu

_L = 6
_NFX = 8
_GAMMA = 0.01


def _sp_sig(a):
    p = jnp.exp(-jnp.abs(a))
    q = 1.0 / (1.0 + p)
    sp = jnp.maximum(a, 0.0) + jnp.log1p(p)
    sig = jnp.where(a >= 0.0, q, 1.0 - q)
    return sp, sig


def _sp(a):
    return jnp.maximum(a, 0.0) + jnp.log1p(jnp.exp(-jnp.abs(a)))


def _rhmc_body(z_ref, v_ref, w_ref, wta_ref, wtc_ref, b_ref, zo_ref, vo_ref):
    f32 = jnp.float32
    W = w_ref[...]
    Wta = wta_ref[...]
    Wtc = wtc_ref[...]
    bias = b_ref[...]

    def mm(x, m):
        return jax.lax.dot_general(
            x.astype(jnp.bfloat16), m, (((1,), (0,)), ((), ())),
            preferred_element_type=f32)

    def chain(z, v):
        a = mm(z, W) + bias
        for _ in range(_L):
            s_z, sig_z = _sp_sig(a)
            u_z = sig_z * (1.0 / s_z)
            vh = v
            for _ in range(_NFX):
                t = sig_z * (vh * vh) - u_z
                vh = vh + mm(t, Wta)
            av = mm(vh, W) + bias
            sp_v, sig_v = _sp_sig(av)
            vh_g = (0.5 * _GAMMA) * vh
            cst = vh_g * s_z + mm(sig_v * (1.0 / sp_v), Wtc)
            zn = z
            for _ in range(_NFX):
                zn = (zn + cst) + _sp(mm(zn, W) + bias) * vh_g
            a = mm(zn, W) + bias
            s_n, sig_n = _sp_sig(a)
            t = sig_n * (vh * vh) - sig_n * (1.0 / s_n)
            v = vh + mm(t, Wta)
            z = zn
        return z, v

    zf, vf = chain(z_ref[...], v_ref[...])
    zo_ref[...] = zf
    vo_ref[...] = vf


@jax.jit
def kernel(z0, v0, W, bias):
    b, d = z0.shape
    block_rows = 256
    grid = (b // block_rows,)
    Wt = W.T
    zf, vf = pl.pallas_call(
        _rhmc_body,
        grid=grid,
        in_specs=[
            pl.BlockSpec((block_rows, d), lambda i: (i, 0)),
            pl.BlockSpec((block_rows, d), lambda i: (i, 0)),
            pl.BlockSpec((d, d), lambda i: (0, 0)),
            pl.BlockSpec((d, d), lambda i: (0, 0)),
            pl.BlockSpec((d, d), lambda i: (0, 0)),
            pl.BlockSpec((1, d), lambda i: (0, 0)),
        ],
        out_specs=[
            pl.BlockSpec((block_rows, d), lambda i: (i, 0)),
            pl.BlockSpec((block_rows, d), lambda i: (i, 0)),
        ],
        out_shape=[
            jax.ShapeDtypeStruct((b, d), jnp.float32),
            jax.ShapeDtypeStruct((b, d), jnp.float32),
        ],
        compiler_params=pltpu.CompilerParams(
            dimension_semantics=("parallel",),
            vmem_limit_bytes=100 * 1024 * 1024,
        ),
    )(z0, v0, W.astype(jnp.bfloat16),
      ((-0.25 * _GAMMA) * Wt).astype(jnp.bfloat16),
      ((-0.5 * _GAMMA) * Wt).astype(jnp.bfloat16),
      bias.reshape(1, d))
    return jnp.stack([zf, vf])


# speedup vs baseline: 1.6706x; 1.6706x over previous
import jax
import jax.numpy as jnp
from jax.experimental import pallas as pl
from jax.experimental.pallas import tpu as pltpu

_L = 6
_NFX = 8
_GAMMA = 0.01


def _sp_sig(a):
    p = jnp.exp(-jnp.abs(a))
    q = 1.0 / (1.0 + p)
    sp = jnp.maximum(a, 0.0) + jnp.log1p(p)
    sig = jnp.where(a >= 0.0, q, 1.0 - q)
    return sp, sig


def _sp(a):
    return jnp.maximum(a, 0.0) + jnp.log1p(jnp.exp(-jnp.abs(a)))


def _rhmc_body(z_ref, v_ref, w_ref, wta_ref, wtc_ref, b_ref, zo_ref, vo_ref):
    f32 = jnp.float32
    bf16 = jnp.bfloat16
    W = w_ref[...]
    Wta = wta_ref[...]
    Wtc = wtc_ref[...]
    bias = b_ref[...]
    bias_b = bias.astype(bf16)

    def mmb(xb, m):
        return jax.lax.dot_general(
            xb, m, (((1,), (0,)), ((), ())),
            preferred_element_type=f32)

    def mm(x, m):
        return mmb(x.astype(bf16), m)

    def sps(a_mm):
        return _sp_sig(a_mm.astype(bf16) + bias_b)

    def chain(z, v):
        a_mm = mmb(z.astype(bf16), W)
        s_b, sig_b = sps(a_mm)
        u_b = sig_b * (1.0 / s_b)
        for _ in range(_L):
            vh = v
            for _ in range(_NFX):
                t = sig_b * (vh * vh).astype(bf16) - u_b
                vh = vh + mmb(t, Wta)
            sp_v, sig_v = sps(mmb(vh.astype(bf16), W))
            vh_g = (0.5 * _GAMMA) * vh
            vh_gb = vh_g.astype(bf16)
            cst = (vh_gb * s_b).astype(f32) + mmb(sig_v * (1.0 / sp_v), Wtc)
            zn = z
            znb = z.astype(bf16)
            for _ in range(_NFX):
                ab = mmb(znb, W).astype(bf16) + bias_b
                zn = (zn + cst) + (_sp(ab) * vh_gb).astype(f32)
                znb = zn.astype(bf16)
            s_b, sig_b = sps(mmb(znb, W))
            u_b = sig_b * (1.0 / s_b)
            t = sig_b * (vh * vh).astype(bf16) - u_b
            v = vh + mmb(t, Wta)
            z = zn
        return z, v

    zf, vf = chain(z_ref[...], v_ref[...])
    zo_ref[...] = zf
    vo_ref[...] = vf


@jax.jit
def kernel(z0, v0, W, bias):
    b, d = z0.shape
    block_rows = 512
    grid = (b // block_rows,)
    Wt = W.T
    zf, vf = pl.pallas_call(
        _rhmc_body,
        grid=grid,
        in_specs=[
            pl.BlockSpec((block_rows, d), lambda i: (i, 0)),
            pl.BlockSpec((block_rows, d), lambda i: (i, 0)),
            pl.BlockSpec((d, d), lambda i: (0, 0)),
            pl.BlockSpec((d, d), lambda i: (0, 0)),
            pl.BlockSpec((d, d), lambda i: (0, 0)),
            pl.BlockSpec((1, d), lambda i: (0, 0)),
        ],
        out_specs=[
            pl.BlockSpec((block_rows, d), lambda i: (i, 0)),
            pl.BlockSpec((block_rows, d), lambda i: (i, 0)),
        ],
        out_shape=[
            jax.ShapeDtypeStruct((b, d), jnp.float32),
            jax.ShapeDtypeStruct((b, d), jnp.float32),
        ],
        compiler_params=pltpu.CompilerParams(
            dimension_semantics=("parallel",),
            vmem_limit_bytes=100 * 1024 * 1024,
        ),
    )(z0, v0, W.astype(jnp.bfloat16),
      ((-0.25 * _GAMMA) * Wt).astype(jnp.bfloat16),
      ((-0.5 * _GAMMA) * Wt).astype(jnp.bfloat16),
      bias.reshape(1, d))
    return jnp.stack([zf, vf])


# speedup vs baseline: 1.7222x; 1.0309x over previous
import jax
import jax.numpy as jnp
from jax.experimental import pallas as pl
from jax.experimental.pallas import tpu as pltpu

_L = 6
_NFX = 8
_GAMMA = 0.01


def _sp_sig(a):
    p = jnp.exp(-jnp.abs(a))
    q = 1.0 / (1.0 + p)
    sp = jnp.maximum(a, 0.0) + jnp.log1p(p)
    sig = jnp.where(a >= 0.0, q, 1.0 - q)
    return sp, sig


def _sp(a):
    return a + jnp.log1p(jnp.exp(-a))


def _rhmc_body(z_ref, v_ref, w_ref, wta_ref, wtc_ref, b_ref, zo_ref, vo_ref):
    f32 = jnp.float32
    bf16 = jnp.bfloat16
    W = w_ref[...]
    Wta = wta_ref[...]
    Wtc = wtc_ref[...]
    bias = b_ref[...]
    bias_b = bias.astype(bf16)

    def mmb(xb, m):
        return jax.lax.dot_general(
            xb, m, (((1,), (0,)), ((), ())),
            preferred_element_type=f32)

    def mm(x, m):
        return mmb(x.astype(bf16), m)

    def sps(a_mm):
        return _sp_sig(a_mm.astype(bf16) + bias_b)

    def chain(z, v):
        a_mm = mmb(z.astype(bf16), W)
        s_b, sig_b = sps(a_mm)
        u_b = sig_b * (1.0 / s_b)
        for _ in range(_L):
            vh = v
            for _ in range(_NFX):
                vhb = vh.astype(bf16)
                t = sig_b * (vhb * vhb) - u_b
                vh = vh + mmb(t, Wta)
            sp_v, sig_v = sps(mmb(vh.astype(bf16), W))
            vh_g = (0.5 * _GAMMA) * vh
            vh_gb = vh_g.astype(bf16)
            cst = (vh_gb * s_b).astype(f32) + mmb(sig_v * (1.0 / sp_v), Wtc)
            zn = z
            znb = z.astype(bf16)
            for _ in range(_NFX):
                ab = mmb(znb, W).astype(bf16) + bias_b
                zn = (zn + cst) + (_sp(ab) * vh_gb).astype(f32)
                znb = zn.astype(bf16)
            s_b, sig_b = sps(mmb(znb, W))
            u_b = sig_b * (1.0 / s_b)
            vhb = vh.astype(bf16)
            t = sig_b * (vhb * vhb) - u_b
            v = vh + mmb(t, Wta)
            z = zn
        return z, v

    zf, vf = chain(z_ref[...], v_ref[...])
    zo_ref[...] = zf
    vo_ref[...] = vf


@jax.jit
def kernel(z0, v0, W, bias):
    b, d = z0.shape
    block_rows = 512
    grid = (b // block_rows,)
    Wt = W.T
    zf, vf = pl.pallas_call(
        _rhmc_body,
        grid=grid,
        in_specs=[
            pl.BlockSpec((block_rows, d), lambda i: (i, 0)),
            pl.BlockSpec((block_rows, d), lambda i: (i, 0)),
            pl.BlockSpec((d, d), lambda i: (0, 0)),
            pl.BlockSpec((d, d), lambda i: (0, 0)),
            pl.BlockSpec((d, d), lambda i: (0, 0)),
            pl.BlockSpec((1, d), lambda i: (0, 0)),
        ],
        out_specs=[
            pl.BlockSpec((block_rows, d), lambda i: (i, 0)),
            pl.BlockSpec((block_rows, d), lambda i: (i, 0)),
        ],
        out_shape=[
            jax.ShapeDtypeStruct((b, d), jnp.float32),
            jax.ShapeDtypeStruct((b, d), jnp.float32),
        ],
        compiler_params=pltpu.CompilerParams(
            dimension_semantics=("parallel",),
            vmem_limit_bytes=100 * 1024 * 1024,
        ),
    )(z0, v0, W.astype(jnp.bfloat16),
      ((-0.25 * _GAMMA) * Wt).astype(jnp.bfloat16),
      ((-0.5 * _GAMMA) * Wt).astype(jnp.bfloat16),
      bias.reshape(1, d))
    return jnp.stack([zf, vf])
